# baseline (speedup 1.0000x reference)
import functools

import jax
import jax.numpy as jnp
from jax import lax
from jax.experimental import pallas as pl
from jax.experimental.pallas import tpu as pltpu
from jax.experimental.pallas import tpu_sc as plsc

NC = 2
NS = 16
NW = NC * NS
CHUNK = 125
CNT_W = 8


def _sc_mesh():
    return plsc.VectorSubcoreMesh(
        core_axis_name="c", subcore_axis_name="s", num_cores=NC, num_subcores=NS
    )


def _make_sc_scatter(n_nodes, e_edges, width, with_count):
    assert e_edges % NW == 0
    per_w = e_edges // NW
    assert per_w % CHUNK == 0
    n_chunks = per_w // CHUNK
    assert n_chunks % 2 == 0
    row_step = (n_nodes // NS) // 8 * 8
    rows_per_sub = n_nodes - row_step * (NS - 1)
    assert rows_per_sub % 8 == 0 and rows_per_sub >= row_step

    out_type = [jax.ShapeDtypeStruct((NC, n_nodes, width), jnp.float32)]
    NBUF = 4
    assert n_chunks % NBUF == 0
    scratch = [
        pltpu.VMEM((n_chunks, CHUNK), jnp.int32),
        pltpu.VMEM((n_chunks, CHUNK), jnp.int32),
    ] + [pltpu.VMEM((CHUNK, width), jnp.float32) for _ in range(NBUF)] + [
        pltpu.VMEM_SHARED((n_nodes, width), jnp.float32),
        pltpu.VMEM_SHARED((n_nodes, width), jnp.float32),
    ] + [pltpu.SemaphoreType.DMA for _ in range(2 * NBUF)]
    if with_count:
        out_type.append(jax.ShapeDtypeStruct((NC, n_nodes, CNT_W), jnp.float32))
        scratch += [
            pltpu.VMEM((CHUNK, CNT_W), jnp.float32),
            pltpu.VMEM_SHARED((n_nodes, CNT_W), jnp.float32),
        ] + [pltpu.SemaphoreType.DMA for _ in range(NBUF)]

    def body(*refs):
        if with_count:
            (p_hbm, ei_hbm, zeros_hbm, zeros1_hbm, ones_hbm,
             s_out, cnt_out, sidx, didx, *rest) = refs
            rows = rest[:NBUF]
            acc, p_sh = rest[NBUF:NBUF + 2]
            sg = rest[NBUF + 2:2 * NBUF + 2]
            ss = rest[2 * NBUF + 2:3 * NBUF + 2]
            ones_v, cacc = rest[3 * NBUF + 2:3 * NBUF + 4]
            sc = rest[3 * NBUF + 4:]
        else:
            (p_hbm, ei_hbm, zeros_hbm,
             s_out, sidx, didx, *rest) = refs
            rows = rest[:NBUF]
            acc, p_sh = rest[NBUF:NBUF + 2]
            sg = rest[NBUF + 2:2 * NBUF + 2]
            ss = rest[2 * NBUF + 2:3 * NBUF + 2]
        cid = lax.axis_index("c")
        sid = lax.axis_index("s")
        wid = sid * NC + cid

        pltpu.sync_copy(ei_hbm.at[0, pl.ds(wid * n_chunks, n_chunks)], sidx)
        pltpu.sync_copy(ei_hbm.at[1, pl.ds(wid * n_chunks, n_chunks)], didx)

        r0 = sid * row_step
        pltpu.sync_copy(zeros_hbm.at[pl.ds(r0, rows_per_sub), :],
                        acc.at[pl.ds(r0, rows_per_sub), :])
        pltpu.sync_copy(p_hbm.at[pl.ds(r0, rows_per_sub), :],
                        p_sh.at[pl.ds(r0, rows_per_sub), :])
        if with_count:
            pltpu.sync_copy(zeros1_hbm.at[pl.ds(r0, rows_per_sub), :],
                            cacc.at[pl.ds(r0, rows_per_sub), :])
            pltpu.sync_copy(ones_hbm, ones_v)
        plsc.subcore_barrier()

        for k in range(NBUF):
            pltpu.async_copy(p_sh.at[sidx.at[k]], rows[k], sg[k])

        def block(j, _):
            for k in range(NBUF):
                c = NBUF * j + k
                pltpu.make_async_copy(p_sh.at[sidx.at[c]], rows[k],
                                      sg[k]).wait()
                pltpu.async_copy(rows[k], acc.at[didx.at[c]], ss[k], add=True)
                if with_count:
                    pltpu.async_copy(ones_v, cacc.at[didx.at[c]], sc[k],
                                     add=True)

                @pl.when(c + NBUF < n_chunks)
                def _():
                    pltpu.make_async_copy(rows[k], acc.at[didx.at[c]],
                                          ss[k]).wait()
                    if with_count:
                        pltpu.make_async_copy(ones_v, cacc.at[didx.at[c]],
                                              sc[k]).wait()
                    pltpu.async_copy(p_sh.at[sidx.at[c + NBUF]], rows[k],
                                     sg[k])
            return 0

        lax.fori_loop(0, n_chunks // NBUF, block, 0)
        for k in range(NBUF):
            c = n_chunks - NBUF + k
            pltpu.make_async_copy(rows[k], acc.at[didx.at[c]], ss[k]).wait()
            if with_count:
                pltpu.make_async_copy(ones_v, cacc.at[didx.at[c]],
                                      sc[k]).wait()
        plsc.subcore_barrier()

        pltpu.sync_copy(acc.at[pl.ds(r0, rows_per_sub), :],
                        s_out.at[cid, pl.ds(r0, rows_per_sub), :])
        if with_count:
            pltpu.sync_copy(cacc.at[pl.ds(r0, rows_per_sub), :],
                            cnt_out.at[cid, pl.ds(r0, rows_per_sub), :])

    return pl.kernel(body, out_type=out_type, mesh=_sc_mesh(),
                     scratch_types=scratch,
                     compiler_params=pltpu.CompilerParams(
                         use_tc_tiling_on_sc=False))


def _blk(shape, idx_fn):
    return pl.BlockSpec(shape, idx_fn)


def _dense1(x, w1a, b1a, w2ax, b2a, nb, rb):
    def body(x_ref, w_ref, b_ref, wx_ref, b2_ref, p_ref, xw_ref):
        xv = x_ref[...]
        p_ref[...] = jnp.maximum(
            jnp.dot(xv, w_ref[...], preferred_element_type=jnp.float32)
            + b_ref[...], 0.0)
        xw_ref[...] = (
            jnp.dot(xv, wx_ref[...], preferred_element_type=jnp.float32)
            + b2_ref[...])

    n, d = x.shape
    k = w1a.shape[1]
    ko = w2ax.shape[1]
    return pl.pallas_call(
        body,
        grid=(nb,),
        in_specs=[_blk((rb, d), lambda i: (i, 0)),
                  _blk(w1a.shape, lambda i: (0, 0)),
                  _blk(b1a.shape, lambda i: (0, 0)),
                  _blk(w2ax.shape, lambda i: (0, 0)),
                  _blk(b2a.shape, lambda i: (0, 0))],
        out_specs=[_blk((rb, k), lambda i: (i, 0)),
                   _blk((rb, ko), lambda i: (i, 0))],
        out_shape=[jax.ShapeDtypeStruct((n, k), jnp.float32),
                   jax.ShapeDtypeStruct((n, ko), jnp.float32)],
    )(x, w1a, b1a, w2ax, b2a)


def _dense2(xw2, s1p, cntp, w2ah, w1b, b1b, w2bx, b2b, nb, rb):
    def body(xw_ref, s_ref, c_ref, wh_ref, w1_ref, b1_ref, wx_ref, b2_ref,
             hw_ref, p2_ref, inv_ref):
        cnt = (c_ref[0] + c_ref[1])[:, 0:1]
        inv = 1.0 / jnp.maximum(cnt, 1.0)
        h = (s_ref[0] + s_ref[1]) * inv
        h1 = jnp.maximum(
            xw_ref[...]
            + jnp.dot(h, wh_ref[...], preferred_element_type=jnp.float32),
            0.0)
        p2_ref[...] = jnp.maximum(
            jnp.dot(h1, w1_ref[...], preferred_element_type=jnp.float32)
            + b1_ref[...], 0.0)
        hw_ref[...] = (
            jnp.dot(h1, wx_ref[...], preferred_element_type=jnp.float32)
            + b2_ref[...])
        inv_ref[...] = inv

    n = xw2.shape[0]
    ks = s1p.shape[2]
    kw = w2bx.shape[1]
    kp = w1b.shape[1]
    return pl.pallas_call(
        body,
        grid=(nb,),
        in_specs=[_blk((rb, xw2.shape[1]), lambda i: (i, 0)),
                  _blk((NC, rb, ks), lambda i: (0, i, 0)),
                  _blk((NC, rb, CNT_W), lambda i: (0, i, 0)),
                  _blk(w2ah.shape, lambda i: (0, 0)),
                  _blk(w1b.shape, lambda i: (0, 0)),
                  _blk(b1b.shape, lambda i: (0, 0)),
                  _blk(w2bx.shape, lambda i: (0, 0)),
                  _blk(b2b.shape, lambda i: (0, 0))],
        out_specs=[_blk((rb, kw), lambda i: (i, 0)),
                   _blk((rb, kp), lambda i: (i, 0)),
                   _blk((rb, 1), lambda i: (i, 0))],
        out_shape=[jax.ShapeDtypeStruct((n, kw), jnp.float32),
                   jax.ShapeDtypeStruct((n, kp), jnp.float32),
                   jax.ShapeDtypeStruct((n, 1), jnp.float32)],
    )(xw2, s1p, cntp, w2ah, w1b, b1b, w2bx, b2b)


def _dense3(hw3, s2p, inv, w2bh, wl, bl, nb, rb):
    def body(hw_ref, s_ref, i_ref, wh_ref, wl_ref, bl_ref, o_ref):
        h = (s_ref[0] + s_ref[1]) * i_ref[...]
        h2 = jnp.maximum(
            hw_ref[...]
            + jnp.dot(h, wh_ref[...], preferred_element_type=jnp.float32),
            0.0)
        o_ref[...] = (
            jnp.dot(h2, wl_ref[...], preferred_element_type=jnp.float32)
            + bl_ref[...])

    n = hw3.shape[0]
    ks = s2p.shape[2]
    return pl.pallas_call(
        body,
        grid=(nb,),
        in_specs=[_blk((rb, hw3.shape[1]), lambda i: (i, 0)),
                  _blk((NC, rb, ks), lambda i: (0, i, 0)),
                  _blk((rb, 1), lambda i: (i, 0)),
                  _blk(w2bh.shape, lambda i: (0, 0)),
                  _blk(wl.shape, lambda i: (0, 0)),
                  _blk(bl.shape, lambda i: (0, 0))],
        out_specs=_blk((rb, 1), lambda i: (i, 0)),
        out_shape=jax.ShapeDtypeStruct((n, 1), jnp.float32),
    )(hw3, s2p, inv, w2bh, wl, bl)


def kernel(x, edge_index, W1a, b1a, W2a, b2a, W1b, b1b, W2b, b2b, Wl, bl):
    n, d = x.shape
    e = edge_index.shape[1]
    ei3 = edge_index.reshape(2, e // CHUNK, CHUNK)
    zeros32 = jnp.zeros((n, 32), jnp.float32)
    zeros16 = jnp.zeros((n, 16), jnp.float32)
    zeros1 = jnp.zeros((n, CNT_W), jnp.float32)
    ones = jnp.ones((CHUNK, CNT_W), jnp.float32)

    nb = 2
    rb = n // nb

    p1, xw2 = _dense1(x, W1a, b1a.reshape(1, -1), W2a[:d], b2a.reshape(1, -1),
                      nb, rb)
    sc1 = _make_sc_scatter(n, e, p1.shape[1], with_count=True)
    s1p, cntp = sc1(p1, ei3, zeros32, zeros1, ones)
    hw3, p2, inv = _dense2(xw2, s1p, cntp, W2a[d:], W1b, b1b.reshape(1, -1),
                           W2b[:W2a.shape[1]], b2b.reshape(1, -1), nb, rb)
    sc2 = _make_sc_scatter(n, e, p2.shape[1], with_count=False)
    s2p = sc2(p2, ei3, zeros16)
    if isinstance(s2p, (list, tuple)):
        s2p = s2p[0]
    return _dense3(hw3, s2p, inv, W2b[W2a.shape[1]:], Wl,
                   bl.reshape(1, -1), nb, rb)


# speedup vs baseline: 1.0291x; 1.0291x over previous
import functools

import jax
import jax.numpy as jnp
from jax import lax
from jax.experimental import pallas as pl
from jax.experimental.pallas import tpu as pltpu
from jax.experimental.pallas import tpu_sc as plsc

NC = 2
NS = 16
NW = NC * NS
CHUNK = 125
CNT_W = 8


def _sc_mesh():
    return plsc.VectorSubcoreMesh(
        core_axis_name="c", subcore_axis_name="s", num_cores=NC, num_subcores=NS
    )


def _make_sc_scatter(n_nodes, e_edges, width, with_count):
    assert e_edges % NW == 0
    per_w = e_edges // NW
    assert per_w % CHUNK == 0
    n_chunks = per_w // CHUNK
    assert n_chunks % 2 == 0
    row_step = (n_nodes // NS) // 8 * 8
    rows_per_sub = n_nodes - row_step * (NS - 1)
    assert rows_per_sub % 8 == 0 and rows_per_sub >= row_step

    out_type = [jax.ShapeDtypeStruct((NC, n_nodes, width), jnp.float32)]
    NBUF = 4
    assert n_chunks % NBUF == 0
    scratch = [
        pltpu.VMEM((n_chunks, CHUNK), jnp.int32),
        pltpu.VMEM((n_chunks, CHUNK), jnp.int32),
    ] + [pltpu.VMEM((CHUNK, width), jnp.float32) for _ in range(NBUF)] + [
        pltpu.VMEM_SHARED((n_nodes, width), jnp.float32),
        pltpu.VMEM_SHARED((n_nodes, width), jnp.float32),
    ] + [pltpu.SemaphoreType.DMA for _ in range(2 * NBUF)]
    if with_count:
        out_type.append(jax.ShapeDtypeStruct((NC, n_nodes, CNT_W), jnp.float32))
        scratch += [
            pltpu.VMEM((CHUNK, CNT_W), jnp.float32),
            pltpu.VMEM_SHARED((n_nodes, CNT_W), jnp.float32),
        ] + [pltpu.SemaphoreType.DMA for _ in range(NBUF)]

    def body(*refs):
        if with_count:
            (p_hbm, ei_hbm, zeros_hbm, zeros1_hbm, ones_hbm,
             s_out, cnt_out, sidx, didx, *rest) = refs
            rows = rest[:NBUF]
            acc, p_sh = rest[NBUF:NBUF + 2]
            sg = rest[NBUF + 2:2 * NBUF + 2]
            ss = rest[2 * NBUF + 2:3 * NBUF + 2]
            ones_v, cacc = rest[3 * NBUF + 2:3 * NBUF + 4]
            sc = rest[3 * NBUF + 4:]
        else:
            (p_hbm, ei_hbm, zeros_hbm,
             s_out, sidx, didx, *rest) = refs
            rows = rest[:NBUF]
            acc, p_sh = rest[NBUF:NBUF + 2]
            sg = rest[NBUF + 2:2 * NBUF + 2]
            ss = rest[2 * NBUF + 2:3 * NBUF + 2]
        cid = lax.axis_index("c")
        sid = lax.axis_index("s")
        wid = sid * NC + cid

        r0 = sid * row_step
        cs = pltpu.async_copy(ei_hbm.at[0, pl.ds(wid * n_chunks, n_chunks)],
                              sidx, sg[0])
        cd = pltpu.async_copy(ei_hbm.at[1, pl.ds(wid * n_chunks, n_chunks)],
                              didx, sg[1])
        cz = pltpu.async_copy(zeros_hbm.at[pl.ds(r0, rows_per_sub), :],
                              acc.at[pl.ds(r0, rows_per_sub), :], sg[2])
        cp = pltpu.async_copy(p_hbm.at[pl.ds(r0, rows_per_sub), :],
                              p_sh.at[pl.ds(r0, rows_per_sub), :], sg[3])
        if with_count:
            cc = pltpu.async_copy(zeros1_hbm.at[pl.ds(r0, rows_per_sub), :],
                                  cacc.at[pl.ds(r0, rows_per_sub), :], ss[0])
            co = pltpu.async_copy(ones_hbm, ones_v, ss[1])
            cc.wait()
            co.wait()
        cs.wait()
        cd.wait()
        cz.wait()
        cp.wait()
        plsc.subcore_barrier()

        for k in range(NBUF):
            pltpu.async_copy(p_sh.at[sidx.at[k]], rows[k], sg[k])

        def block(j, _):
            for k in range(NBUF):
                c = NBUF * j + k
                pltpu.make_async_copy(p_sh.at[sidx.at[c]], rows[k],
                                      sg[k]).wait()
                pltpu.async_copy(rows[k], acc.at[didx.at[c]], ss[k], add=True)
                if with_count:
                    pltpu.async_copy(ones_v, cacc.at[didx.at[c]], sc[k],
                                     add=True)

                @pl.when(c + NBUF < n_chunks)
                def _():
                    pltpu.make_async_copy(rows[k], acc.at[didx.at[c]],
                                          ss[k]).wait()
                    if with_count:
                        pltpu.make_async_copy(ones_v, cacc.at[didx.at[c]],
                                              sc[k]).wait()
                    pltpu.async_copy(p_sh.at[sidx.at[c + NBUF]], rows[k],
                                     sg[k])
            return 0

        lax.fori_loop(0, n_chunks // NBUF, block, 0)
        for k in range(NBUF):
            c = n_chunks - NBUF + k
            pltpu.make_async_copy(rows[k], acc.at[didx.at[c]], ss[k]).wait()
            if with_count:
                pltpu.make_async_copy(ones_v, cacc.at[didx.at[c]],
                                      sc[k]).wait()
        plsc.subcore_barrier()

        pltpu.sync_copy(acc.at[pl.ds(r0, rows_per_sub), :],
                        s_out.at[cid, pl.ds(r0, rows_per_sub), :])
        if with_count:
            pltpu.sync_copy(cacc.at[pl.ds(r0, rows_per_sub), :],
                            cnt_out.at[cid, pl.ds(r0, rows_per_sub), :])

    return pl.kernel(body, out_type=out_type, mesh=_sc_mesh(),
                     scratch_types=scratch,
                     compiler_params=pltpu.CompilerParams(
                         use_tc_tiling_on_sc=False))


def _blk(shape, idx_fn):
    return pl.BlockSpec(shape, idx_fn)


def _dense1(x, w1a, b1a, w2ax, b2a, nb, rb):
    def body(x_ref, w_ref, b_ref, wx_ref, b2_ref, p_ref, xw_ref):
        xv = x_ref[...]
        p_ref[...] = jnp.maximum(
            jnp.dot(xv, w_ref[...], preferred_element_type=jnp.float32)
            + b_ref[...], 0.0)
        xw_ref[...] = (
            jnp.dot(xv, wx_ref[...], preferred_element_type=jnp.float32)
            + b2_ref[...])

    n, d = x.shape
    k = w1a.shape[1]
    ko = w2ax.shape[1]
    return pl.pallas_call(
        body,
        grid=(nb,),
        in_specs=[_blk((rb, d), lambda i: (i, 0)),
                  _blk(w1a.shape, lambda i: (0, 0)),
                  _blk(b1a.shape, lambda i: (0, 0)),
                  _blk(w2ax.shape, lambda i: (0, 0)),
                  _blk(b2a.shape, lambda i: (0, 0))],
        out_specs=[_blk((rb, k), lambda i: (i, 0)),
                   _blk((rb, ko), lambda i: (i, 0))],
        out_shape=[jax.ShapeDtypeStruct((n, k), jnp.float32),
                   jax.ShapeDtypeStruct((n, ko), jnp.float32)],
    )(x, w1a, b1a, w2ax, b2a)


def _dense2(xw2, s1p, cntp, w2ah, w1b, b1b, w2bx, b2b, nb, rb):
    def body(xw_ref, s_ref, c_ref, wh_ref, w1_ref, b1_ref, wx_ref, b2_ref,
             hw_ref, p2_ref, inv_ref):
        cnt = (c_ref[0] + c_ref[1])[:, 0:1]
        inv = 1.0 / jnp.maximum(cnt, 1.0)
        h = (s_ref[0] + s_ref[1]) * inv
        h1 = jnp.maximum(
            xw_ref[...]
            + jnp.dot(h, wh_ref[...], preferred_element_type=jnp.float32),
            0.0)
        p2_ref[...] = jnp.maximum(
            jnp.dot(h1, w1_ref[...], preferred_element_type=jnp.float32)
            + b1_ref[...], 0.0)
        hw_ref[...] = (
            jnp.dot(h1, wx_ref[...], preferred_element_type=jnp.float32)
            + b2_ref[...])
        inv_ref[...] = inv

    n = xw2.shape[0]
    ks = s1p.shape[2]
    kw = w2bx.shape[1]
    kp = w1b.shape[1]
    return pl.pallas_call(
        body,
        grid=(nb,),
        in_specs=[_blk((rb, xw2.shape[1]), lambda i: (i, 0)),
                  _blk((NC, rb, ks), lambda i: (0, i, 0)),
                  _blk((NC, rb, CNT_W), lambda i: (0, i, 0)),
                  _blk(w2ah.shape, lambda i: (0, 0)),
                  _blk(w1b.shape, lambda i: (0, 0)),
                  _blk(b1b.shape, lambda i: (0, 0)),
                  _blk(w2bx.shape, lambda i: (0, 0)),
                  _blk(b2b.shape, lambda i: (0, 0))],
        out_specs=[_blk((rb, kw), lambda i: (i, 0)),
                   _blk((rb, kp), lambda i: (i, 0)),
                   _blk((rb, 1), lambda i: (i, 0))],
        out_shape=[jax.ShapeDtypeStruct((n, kw), jnp.float32),
                   jax.ShapeDtypeStruct((n, kp), jnp.float32),
                   jax.ShapeDtypeStruct((n, 1), jnp.float32)],
    )(xw2, s1p, cntp, w2ah, w1b, b1b, w2bx, b2b)


def _dense3(hw3, s2p, inv, w2bh, wl, bl, nb, rb):
    def body(hw_ref, s_ref, i_ref, wh_ref, wl_ref, bl_ref, o_ref):
        h = (s_ref[0] + s_ref[1]) * i_ref[...]
        h2 = jnp.maximum(
            hw_ref[...]
            + jnp.dot(h, wh_ref[...], preferred_element_type=jnp.float32),
            0.0)
        o_ref[...] = (
            jnp.dot(h2, wl_ref[...], preferred_element_type=jnp.float32)
            + bl_ref[...])

    n = hw3.shape[0]
    ks = s2p.shape[2]
    return pl.pallas_call(
        body,
        grid=(nb,),
        in_specs=[_blk((rb, hw3.shape[1]), lambda i: (i, 0)),
                  _blk((NC, rb, ks), lambda i: (0, i, 0)),
                  _blk((rb, 1), lambda i: (i, 0)),
                  _blk(w2bh.shape, lambda i: (0, 0)),
                  _blk(wl.shape, lambda i: (0, 0)),
                  _blk(bl.shape, lambda i: (0, 0))],
        out_specs=_blk((rb, 1), lambda i: (i, 0)),
        out_shape=jax.ShapeDtypeStruct((n, 1), jnp.float32),
    )(hw3, s2p, inv, w2bh, wl, bl)


def kernel(x, edge_index, W1a, b1a, W2a, b2a, W1b, b1b, W2b, b2b, Wl, bl):
    n, d = x.shape
    e = edge_index.shape[1]
    ei3 = edge_index.reshape(2, e // CHUNK, CHUNK)
    zeros32 = jnp.zeros((n, 32), jnp.float32)
    zeros16 = jnp.zeros((n, 16), jnp.float32)
    zeros1 = jnp.zeros((n, CNT_W), jnp.float32)
    ones = jnp.ones((CHUNK, CNT_W), jnp.float32)

    nb = 2
    rb = n // nb

    p1, xw2 = _dense1(x, W1a, b1a.reshape(1, -1), W2a[:d], b2a.reshape(1, -1),
                      nb, rb)
    sc1 = _make_sc_scatter(n, e, p1.shape[1], with_count=True)
    s1p, cntp = sc1(p1, ei3, zeros32, zeros1, ones)
    hw3, p2, inv = _dense2(xw2, s1p, cntp, W2a[d:], W1b, b1b.reshape(1, -1),
                           W2b[:W2a.shape[1]], b2b.reshape(1, -1), nb, rb)
    sc2 = _make_sc_scatter(n, e, p2.shape[1], with_count=False)
    s2p = sc2(p2, ei3, zeros16)
    if isinstance(s2p, (list, tuple)):
        s2p = s2p[0]
    return _dense3(hw3, s2p, inv, W2b[W2a.shape[1]:], Wl,
                   bl.reshape(1, -1), nb, rb)


# speedup vs baseline: 1.0385x; 1.0092x over previous
import functools

import jax
import jax.numpy as jnp
from jax import lax
from jax.experimental import pallas as pl
from jax.experimental.pallas import tpu as pltpu
from jax.experimental.pallas import tpu_sc as plsc

NC = 2
NS = 16
NW = NC * NS
CHUNK = 125
CNT_W = 8


def _sc_mesh():
    return plsc.VectorSubcoreMesh(
        core_axis_name="c", subcore_axis_name="s", num_cores=NC, num_subcores=NS
    )


def _make_sc_scatter(n_nodes, e_edges, width, with_count):
    assert e_edges % NW == 0
    per_w = e_edges // NW
    assert per_w % CHUNK == 0
    n_chunks = per_w // CHUNK
    assert n_chunks % 2 == 0
    row_step = (n_nodes // NS) // 8 * 8
    rows_per_sub = n_nodes - row_step * (NS - 1)
    assert rows_per_sub % 8 == 0 and rows_per_sub >= row_step

    out_type = [jax.ShapeDtypeStruct((NC, n_nodes, width), jnp.float32)]
    NBUF = 4
    assert n_chunks % NBUF == 0
    scratch = [
        pltpu.VMEM((n_chunks, CHUNK), jnp.int32),
        pltpu.VMEM((n_chunks, CHUNK), jnp.int32),
    ] + [pltpu.VMEM((CHUNK, width), jnp.float32) for _ in range(NBUF)] + [
        pltpu.VMEM_SHARED((n_nodes, width), jnp.float32),
        pltpu.VMEM_SHARED((n_nodes, width), jnp.float32),
    ] + [pltpu.SemaphoreType.DMA for _ in range(2 * NBUF)]
    if with_count:
        out_type.append(jax.ShapeDtypeStruct((NC, n_nodes, CNT_W), jnp.float32))
        scratch += [
            pltpu.VMEM((CHUNK, CNT_W), jnp.float32),
            pltpu.VMEM_SHARED((n_nodes, CNT_W), jnp.float32),
        ] + [pltpu.SemaphoreType.DMA for _ in range(NBUF)]

    def body(*refs):
        if with_count:
            (p_hbm, ei_hbm, zeros_hbm, zeros1_hbm, ones_hbm,
             s_out, cnt_out, sidx, didx, *rest) = refs
            rows = rest[:NBUF]
            acc, p_sh = rest[NBUF:NBUF + 2]
            sg = rest[NBUF + 2:2 * NBUF + 2]
            ss = rest[2 * NBUF + 2:3 * NBUF + 2]
            ones_v, cacc = rest[3 * NBUF + 2:3 * NBUF + 4]
            sc = rest[3 * NBUF + 4:]
        else:
            (p_hbm, ei_hbm, zeros_hbm,
             s_out, sidx, didx, *rest) = refs
            rows = rest[:NBUF]
            acc, p_sh = rest[NBUF:NBUF + 2]
            sg = rest[NBUF + 2:2 * NBUF + 2]
            ss = rest[2 * NBUF + 2:3 * NBUF + 2]
        cid = lax.axis_index("c")
        sid = lax.axis_index("s")
        wid = sid * NC + cid

        r0 = sid * row_step
        cs = pltpu.async_copy(ei_hbm.at[0, pl.ds(wid * n_chunks, n_chunks)],
                              sidx, sg[0])
        cd = pltpu.async_copy(ei_hbm.at[1, pl.ds(wid * n_chunks, n_chunks)],
                              didx, sg[1])
        cz = pltpu.async_copy(zeros_hbm.at[pl.ds(r0, rows_per_sub), :],
                              acc.at[pl.ds(r0, rows_per_sub), :], sg[2])
        cp = pltpu.async_copy(p_hbm.at[pl.ds(r0, rows_per_sub), :],
                              p_sh.at[pl.ds(r0, rows_per_sub), :], sg[3])
        if with_count:
            cc = pltpu.async_copy(zeros1_hbm.at[pl.ds(r0, rows_per_sub), :],
                                  cacc.at[pl.ds(r0, rows_per_sub), :], ss[0])
            co = pltpu.async_copy(ones_hbm, ones_v, ss[1])
            cc.wait()
            co.wait()
        cs.wait()
        cd.wait()
        cz.wait()
        cp.wait()
        plsc.subcore_barrier()

        for k in range(NBUF):
            pltpu.async_copy(p_sh.at[sidx.at[k]], rows[k], sg[k])

        def block(j, _):
            for k in range(NBUF):
                c = NBUF * j + k
                pltpu.make_async_copy(p_sh.at[sidx.at[c]], rows[k],
                                      sg[k]).wait()
                pltpu.async_copy(rows[k], acc.at[didx.at[c]], ss[k], add=True)
                if with_count:
                    pltpu.async_copy(ones_v, cacc.at[didx.at[c]], sc[k],
                                     add=True)

                @pl.when(c + NBUF < n_chunks)
                def _():
                    pltpu.make_async_copy(rows[k], acc.at[didx.at[c]],
                                          ss[k]).wait()
                    if with_count:
                        pltpu.make_async_copy(ones_v, cacc.at[didx.at[c]],
                                              sc[k]).wait()
                    pltpu.async_copy(p_sh.at[sidx.at[c + NBUF]], rows[k],
                                     sg[k])
            return 0

        lax.fori_loop(0, n_chunks // NBUF, block, 0)
        for k in range(NBUF):
            c = n_chunks - NBUF + k
            pltpu.make_async_copy(rows[k], acc.at[didx.at[c]], ss[k]).wait()
            if with_count:
                pltpu.make_async_copy(ones_v, cacc.at[didx.at[c]],
                                      sc[k]).wait()
        plsc.subcore_barrier()

        pltpu.sync_copy(acc.at[pl.ds(r0, rows_per_sub), :],
                        s_out.at[cid, pl.ds(r0, rows_per_sub), :])
        if with_count:
            pltpu.sync_copy(cacc.at[pl.ds(r0, rows_per_sub), :],
                            cnt_out.at[cid, pl.ds(r0, rows_per_sub), :])

    return pl.kernel(body, out_type=out_type, mesh=_sc_mesh(),
                     scratch_types=scratch,
                     compiler_params=pltpu.CompilerParams(
                         use_tc_tiling_on_sc=False))


def _blk(shape, idx_fn):
    return pl.BlockSpec(shape, idx_fn)


def _dense1(x, w1a, b1a, nb, rb):
    def body(x_ref, w_ref, b_ref, o_ref):
        o_ref[...] = jnp.maximum(
            jnp.dot(x_ref[...], w_ref[...], preferred_element_type=jnp.float32)
            + b_ref[...], 0.0)

    n, d = x.shape
    k = w1a.shape[1]
    return pl.pallas_call(
        body,
        grid=(nb,),
        in_specs=[_blk((rb, d), lambda i: (i, 0)),
                  _blk(w1a.shape, lambda i: (0, 0)),
                  _blk(b1a.shape, lambda i: (0, 0))],
        out_specs=_blk((rb, k), lambda i: (i, 0)),
        out_shape=jax.ShapeDtypeStruct((n, k), jnp.float32),
    )(x, w1a, b1a)


def _dense2(x, s1p, cntp, w2a, b2a, w1b, b1b, nb, rb):
    def body(x_ref, s_ref, c_ref, w2_ref, b2_ref, w1_ref, b1_ref,
             h1_ref, p2_ref, inv_ref):
        cnt = (c_ref[0] + c_ref[1])[:, 0:1]
        inv = 1.0 / jnp.maximum(cnt, 1.0)
        h = (s_ref[0] + s_ref[1]) * inv
        din = x_ref.shape[1]
        h1 = jnp.maximum(
            jnp.dot(x_ref[...], w2_ref[:din], preferred_element_type=jnp.float32)
            + jnp.dot(h, w2_ref[din:], preferred_element_type=jnp.float32)
            + b2_ref[...], 0.0)
        h1_ref[...] = h1
        p2_ref[...] = jnp.maximum(
            jnp.dot(h1, w1_ref[...], preferred_element_type=jnp.float32)
            + b1_ref[...], 0.0)
        inv_ref[...] = inv

    n, d = x.shape
    ks = s1p.shape[2]
    ko = w2a.shape[1]
    kp = w1b.shape[1]
    return pl.pallas_call(
        body,
        grid=(nb,),
        in_specs=[_blk((rb, d), lambda i: (i, 0)),
                  _blk((NC, rb, ks), lambda i: (0, i, 0)),
                  _blk((NC, rb, CNT_W), lambda i: (0, i, 0)),
                  _blk(w2a.shape, lambda i: (0, 0)),
                  _blk(b2a.shape, lambda i: (0, 0)),
                  _blk(w1b.shape, lambda i: (0, 0)),
                  _blk(b1b.shape, lambda i: (0, 0))],
        out_specs=[_blk((rb, ko), lambda i: (i, 0)),
                   _blk((rb, kp), lambda i: (i, 0)),
                   _blk((rb, 1), lambda i: (i, 0))],
        out_shape=[jax.ShapeDtypeStruct((n, ko), jnp.float32),
                   jax.ShapeDtypeStruct((n, kp), jnp.float32),
                   jax.ShapeDtypeStruct((n, 1), jnp.float32)],
    )(x, s1p, cntp, w2a, b2a, w1b, b1b)


def _dense3(h1, s2p, inv, w2b, b2b, wl, bl, nb, rb):
    def body(h1_ref, s_ref, i_ref, w2_ref, b2_ref, wl_ref, bl_ref, o_ref):
        h = (s_ref[0] + s_ref[1]) * i_ref[...]
        din = h1_ref.shape[1]
        h2 = jnp.maximum(
            jnp.dot(h1_ref[...], w2_ref[:din], preferred_element_type=jnp.float32)
            + jnp.dot(h, w2_ref[din:], preferred_element_type=jnp.float32)
            + b2_ref[...], 0.0)
        o_ref[...] = (
            jnp.dot(h2, wl_ref[...], preferred_element_type=jnp.float32)
            + bl_ref[...])

    n, d = h1.shape
    ks = s2p.shape[2]
    return pl.pallas_call(
        body,
        grid=(nb,),
        in_specs=[_blk((rb, d), lambda i: (i, 0)),
                  _blk((NC, rb, ks), lambda i: (0, i, 0)),
                  _blk((rb, 1), lambda i: (i, 0)),
                  _blk(w2b.shape, lambda i: (0, 0)),
                  _blk(b2b.shape, lambda i: (0, 0)),
                  _blk(wl.shape, lambda i: (0, 0)),
                  _blk(bl.shape, lambda i: (0, 0))],
        out_specs=_blk((rb, 1), lambda i: (i, 0)),
        out_shape=jax.ShapeDtypeStruct((n, 1), jnp.float32),
    )(h1, s2p, inv, w2b, b2b, wl, bl)


def kernel(x, edge_index, W1a, b1a, W2a, b2a, W1b, b1b, W2b, b2b, Wl, bl):
    n, d = x.shape
    e = edge_index.shape[1]
    ei3 = edge_index.reshape(2, e // CHUNK, CHUNK)
    zeros32 = jnp.zeros((n, 32), jnp.float32)
    zeros16 = jnp.zeros((n, 16), jnp.float32)
    zeros1 = jnp.zeros((n, CNT_W), jnp.float32)
    ones = jnp.ones((CHUNK, CNT_W), jnp.float32)

    nb = 2
    rb = n // nb

    p1 = _dense1(x, W1a, b1a.reshape(1, -1), nb, rb)
    sc1 = _make_sc_scatter(n, e, p1.shape[1], with_count=True)
    s1p, cntp = sc1(p1, ei3, zeros32, zeros1, ones)
    h1, p2, inv = _dense2(x, s1p, cntp, W2a, b2a.reshape(1, -1),
                          W1b, b1b.reshape(1, -1), nb, rb)
    sc2 = _make_sc_scatter(n, e, p2.shape[1], with_count=False)
    s2p = sc2(p2, ei3, zeros16)
    if isinstance(s2p, (list, tuple)):
        s2p = s2p[0]
    return _dense3(h1, s2p, inv, W2b, b2b.reshape(1, -1), Wl,
                   bl.reshape(1, -1), nb, rb)


# speedup vs baseline: 1.0408x; 1.0022x over previous
import functools

import jax
import jax.numpy as jnp
from jax import lax
from jax.experimental import pallas as pl
from jax.experimental.pallas import tpu as pltpu
from jax.experimental.pallas import tpu_sc as plsc

NC = 2
NS = 16
NW = NC * NS
CHUNK = 125
CNT_W = 8


def _sc_mesh():
    return plsc.VectorSubcoreMesh(
        core_axis_name="c", subcore_axis_name="s", num_cores=NC, num_subcores=NS
    )


def _make_sc_scatter(n_nodes, e_edges, width, with_count):
    assert e_edges % NW == 0
    per_w = e_edges // NW
    assert per_w % CHUNK == 0
    n_chunks = per_w // CHUNK
    assert n_chunks % 2 == 0
    row_step = (n_nodes // NS) // 8 * 8
    rows_per_sub = n_nodes - row_step * (NS - 1)
    assert rows_per_sub % 8 == 0 and rows_per_sub >= row_step

    out_type = [jax.ShapeDtypeStruct((NC, n_nodes, width), jnp.float32)]
    NBUF = 8
    assert n_chunks % NBUF == 0
    scratch = [
        pltpu.VMEM((n_chunks, CHUNK), jnp.int32),
        pltpu.VMEM((n_chunks, CHUNK), jnp.int32),
    ] + [pltpu.VMEM((CHUNK, width), jnp.float32) for _ in range(NBUF)] + [
        pltpu.VMEM_SHARED((n_nodes, width), jnp.float32),
        pltpu.VMEM_SHARED((n_nodes, width), jnp.float32),
    ] + [pltpu.SemaphoreType.DMA for _ in range(2 * NBUF)]
    if with_count:
        out_type.append(jax.ShapeDtypeStruct((NC, n_nodes, CNT_W), jnp.float32))
        scratch += [
            pltpu.VMEM((CHUNK, CNT_W), jnp.float32),
            pltpu.VMEM_SHARED((n_nodes, CNT_W), jnp.float32),
        ] + [pltpu.SemaphoreType.DMA for _ in range(NBUF)]

    def body(*refs):
        if with_count:
            (p_hbm, ei_hbm, zeros_hbm, zeros1_hbm, ones_hbm,
             s_out, cnt_out, sidx, didx, *rest) = refs
            rows = rest[:NBUF]
            acc, p_sh = rest[NBUF:NBUF + 2]
            sg = rest[NBUF + 2:2 * NBUF + 2]
            ss = rest[2 * NBUF + 2:3 * NBUF + 2]
            ones_v, cacc = rest[3 * NBUF + 2:3 * NBUF + 4]
            sc = rest[3 * NBUF + 4:]
        else:
            (p_hbm, ei_hbm, zeros_hbm,
             s_out, sidx, didx, *rest) = refs
            rows = rest[:NBUF]
            acc, p_sh = rest[NBUF:NBUF + 2]
            sg = rest[NBUF + 2:2 * NBUF + 2]
            ss = rest[2 * NBUF + 2:3 * NBUF + 2]
        cid = lax.axis_index("c")
        sid = lax.axis_index("s")
        wid = sid * NC + cid

        r0 = sid * row_step
        cs = pltpu.async_copy(ei_hbm.at[0, pl.ds(wid * n_chunks, n_chunks)],
                              sidx, sg[0])
        cd = pltpu.async_copy(ei_hbm.at[1, pl.ds(wid * n_chunks, n_chunks)],
                              didx, sg[1])
        cz = pltpu.async_copy(zeros_hbm.at[pl.ds(r0, rows_per_sub), :],
                              acc.at[pl.ds(r0, rows_per_sub), :], sg[2])
        cp = pltpu.async_copy(p_hbm.at[pl.ds(r0, rows_per_sub), :],
                              p_sh.at[pl.ds(r0, rows_per_sub), :], sg[3])
        if with_count:
            cc = pltpu.async_copy(zeros1_hbm.at[pl.ds(r0, rows_per_sub), :],
                                  cacc.at[pl.ds(r0, rows_per_sub), :], ss[0])
            co = pltpu.async_copy(ones_hbm, ones_v, ss[1])
            cc.wait()
            co.wait()
        cs.wait()
        cd.wait()
        cz.wait()
        cp.wait()
        plsc.subcore_barrier()

        for k in range(NBUF):
            pltpu.async_copy(p_sh.at[sidx.at[k]], rows[k], sg[k])

        def block(j, _):
            for k in range(NBUF):
                c = NBUF * j + k
                pltpu.make_async_copy(p_sh.at[sidx.at[c]], rows[k],
                                      sg[k]).wait()
                pltpu.async_copy(rows[k], acc.at[didx.at[c]], ss[k], add=True)
                if with_count:
                    pltpu.async_copy(ones_v, cacc.at[didx.at[c]], sc[k],
                                     add=True)

                @pl.when(c + NBUF < n_chunks)
                def _():
                    pltpu.make_async_copy(rows[k], acc.at[didx.at[c]],
                                          ss[k]).wait()
                    if with_count:
                        pltpu.make_async_copy(ones_v, cacc.at[didx.at[c]],
                                              sc[k]).wait()
                    pltpu.async_copy(p_sh.at[sidx.at[c + NBUF]], rows[k],
                                     sg[k])
            return 0

        lax.fori_loop(0, n_chunks // NBUF, block, 0)
        for k in range(NBUF):
            c = n_chunks - NBUF + k
            pltpu.make_async_copy(rows[k], acc.at[didx.at[c]], ss[k]).wait()
            if with_count:
                pltpu.make_async_copy(ones_v, cacc.at[didx.at[c]],
                                      sc[k]).wait()
        plsc.subcore_barrier()

        pltpu.sync_copy(acc.at[pl.ds(r0, rows_per_sub), :],
                        s_out.at[cid, pl.ds(r0, rows_per_sub), :])
        if with_count:
            pltpu.sync_copy(cacc.at[pl.ds(r0, rows_per_sub), :],
                            cnt_out.at[cid, pl.ds(r0, rows_per_sub), :])

    return pl.kernel(body, out_type=out_type, mesh=_sc_mesh(),
                     scratch_types=scratch,
                     compiler_params=pltpu.CompilerParams(
                         use_tc_tiling_on_sc=False))


def _blk(shape, idx_fn):
    return pl.BlockSpec(shape, idx_fn)


def _dense1(x, w1a, b1a, nb, rb):
    def body(x_ref, w_ref, b_ref, o_ref):
        o_ref[...] = jnp.maximum(
            jnp.dot(x_ref[...], w_ref[...], preferred_element_type=jnp.float32)
            + b_ref[...], 0.0)

    n, d = x.shape
    k = w1a.shape[1]
    return pl.pallas_call(
        body,
        grid=(nb,),
        in_specs=[_blk((rb, d), lambda i: (i, 0)),
                  _blk(w1a.shape, lambda i: (0, 0)),
                  _blk(b1a.shape, lambda i: (0, 0))],
        out_specs=_blk((rb, k), lambda i: (i, 0)),
        out_shape=jax.ShapeDtypeStruct((n, k), jnp.float32),
    )(x, w1a, b1a)


def _dense2(x, s1p, cntp, w2a, b2a, w1b, b1b, nb, rb):
    def body(x_ref, s_ref, c_ref, w2_ref, b2_ref, w1_ref, b1_ref,
             h1_ref, p2_ref, inv_ref):
        cnt = (c_ref[0] + c_ref[1])[:, 0:1]
        inv = 1.0 / jnp.maximum(cnt, 1.0)
        h = (s_ref[0] + s_ref[1]) * inv
        din = x_ref.shape[1]
        h1 = jnp.maximum(
            jnp.dot(x_ref[...], w2_ref[:din], preferred_element_type=jnp.float32)
            + jnp.dot(h, w2_ref[din:], preferred_element_type=jnp.float32)
            + b2_ref[...], 0.0)
        h1_ref[...] = h1
        p2_ref[...] = jnp.maximum(
            jnp.dot(h1, w1_ref[...], preferred_element_type=jnp.float32)
            + b1_ref[...], 0.0)
        inv_ref[...] = inv

    n, d = x.shape
    ks = s1p.shape[2]
    ko = w2a.shape[1]
    kp = w1b.shape[1]
    return pl.pallas_call(
        body,
        grid=(nb,),
        in_specs=[_blk((rb, d), lambda i: (i, 0)),
                  _blk((NC, rb, ks), lambda i: (0, i, 0)),
                  _blk((NC, rb, CNT_W), lambda i: (0, i, 0)),
                  _blk(w2a.shape, lambda i: (0, 0)),
                  _blk(b2a.shape, lambda i: (0, 0)),
                  _blk(w1b.shape, lambda i: (0, 0)),
                  _blk(b1b.shape, lambda i: (0, 0))],
        out_specs=[_blk((rb, ko), lambda i: (i, 0)),
                   _blk((rb, kp), lambda i: (i, 0)),
                   _blk((rb, 1), lambda i: (i, 0))],
        out_shape=[jax.ShapeDtypeStruct((n, ko), jnp.float32),
                   jax.ShapeDtypeStruct((n, kp), jnp.float32),
                   jax.ShapeDtypeStruct((n, 1), jnp.float32)],
    )(x, s1p, cntp, w2a, b2a, w1b, b1b)


def _dense3(h1, s2p, inv, w2b, b2b, wl, bl, nb, rb):
    def body(h1_ref, s_ref, i_ref, w2_ref, b2_ref, wl_ref, bl_ref, o_ref):
        h = (s_ref[0] + s_ref[1]) * i_ref[...]
        din = h1_ref.shape[1]
        h2 = jnp.maximum(
            jnp.dot(h1_ref[...], w2_ref[:din], preferred_element_type=jnp.float32)
            + jnp.dot(h, w2_ref[din:], preferred_element_type=jnp.float32)
            + b2_ref[...], 0.0)
        o_ref[...] = (
            jnp.dot(h2, wl_ref[...], preferred_element_type=jnp.float32)
            + bl_ref[...])

    n, d = h1.shape
    ks = s2p.shape[2]
    return pl.pallas_call(
        body,
        grid=(nb,),
        in_specs=[_blk((rb, d), lambda i: (i, 0)),
                  _blk((NC, rb, ks), lambda i: (0, i, 0)),
                  _blk((rb, 1), lambda i: (i, 0)),
                  _blk(w2b.shape, lambda i: (0, 0)),
                  _blk(b2b.shape, lambda i: (0, 0)),
                  _blk(wl.shape, lambda i: (0, 0)),
                  _blk(bl.shape, lambda i: (0, 0))],
        out_specs=_blk((rb, 1), lambda i: (i, 0)),
        out_shape=jax.ShapeDtypeStruct((n, 1), jnp.float32),
    )(h1, s2p, inv, w2b, b2b, wl, bl)


def kernel(x, edge_index, W1a, b1a, W2a, b2a, W1b, b1b, W2b, b2b, Wl, bl):
    n, d = x.shape
    e = edge_index.shape[1]
    ei3 = edge_index.reshape(2, e // CHUNK, CHUNK)
    zeros32 = jnp.zeros((n, 32), jnp.float32)
    zeros16 = jnp.zeros((n, 16), jnp.float32)
    zeros1 = jnp.zeros((n, CNT_W), jnp.float32)
    ones = jnp.ones((CHUNK, CNT_W), jnp.float32)

    nb = 2
    rb = n // nb

    p1 = _dense1(x, W1a, b1a.reshape(1, -1), nb, rb)
    sc1 = _make_sc_scatter(n, e, p1.shape[1], with_count=True)
    s1p, cntp = sc1(p1, ei3, zeros32, zeros1, ones)
    h1, p2, inv = _dense2(x, s1p, cntp, W2a, b2a.reshape(1, -1),
                          W1b, b1b.reshape(1, -1), nb, rb)
    sc2 = _make_sc_scatter(n, e, p2.shape[1], with_count=False)
    s2p = sc2(p2, ei3, zeros16)
    if isinstance(s2p, (list, tuple)):
        s2p = s2p[0]
    return _dense3(h1, s2p, inv, W2b, b2b.reshape(1, -1), Wl,
                   bl.reshape(1, -1), nb, rb)
